# Initial kernel scaffold; baseline (speedup 1.0000x reference)
import jax
import jax.numpy as jnp
from jax.experimental import pallas as pl


def kernel(x, edge_index, edge_attr, batch, w1, w2, w3, gate_w, gate_b, nn_w, nn_b):
    raise NotImplementedError("write your pallas kernel here")


# speedup vs baseline: 7.5736x; 7.5736x over previous
import functools

import jax
import jax.numpy as jnp
from jax import lax
from jax.experimental import pallas as pl
from jax.experimental.pallas import tpu as pltpu
from jax.experimental.pallas import tpu_sc as plsc

N = 10000
H = 128
G = 64
CH = 128
NC = 2
NS = 16
NW = NC * NS
NP = 10240
ZROWS = NP // NS


def _rsqrt_approx(d):
    ib = lax.bitcast_convert_type(d, jnp.int32)
    y = lax.bitcast_convert_type(jnp.int32(0x5F3759DF) - (ib >> 1), jnp.float32)
    hd = 0.5 * d
    y = y * (1.5 - hd * y * y)
    y = y * (1.5 - hd * y * y)
    y = y * (1.5 - hd * y * y)
    return y


def _make_sc_kernels(nchunk, cpw):
    mesh = plsc.VectorSubcoreMesh(core_axis_name="c", subcore_axis_name="s")
    sc_params = pltpu.CompilerParams(needs_layout_passes=False)

    @functools.partial(
        pl.kernel,
        out_type=jax.ShapeDtypeStruct((NC, NP), jnp.float32),
        mesh=mesh,
        compiler_params=sc_params,
        scratch_types=[
            pltpu.VMEM((CH,), jnp.int32),
            pltpu.VMEM((CH,), jnp.float32),
            pltpu.VMEM((ZROWS,), jnp.float32),
            pltpu.VMEM_SHARED((NP,), jnp.float32),
        ],
    )
    def deg_kernel(col_hbm, w_hbm, out_hbm, idx_v, val_v, zbuf, sh_deg):
        cid = lax.axis_index("c")
        sid = lax.axis_index("s")

        def zb(i, _):
            zbuf[pl.ds(i * 16, 16)] = jnp.zeros((16,), jnp.float32)
            return 0

        lax.fori_loop(0, ZROWS // 16, zb, 0)
        pltpu.sync_copy(zbuf, sh_deg.at[pl.ds(sid * ZROWS, ZROWS)])
        plsc.subcore_barrier()

        base = cid * (nchunk // NC) + sid * cpw

        def body(k, _):
            pltpu.sync_copy(col_hbm.at[base + k], idx_v)
            pltpu.sync_copy(w_hbm.at[base + k], val_v)
            pltpu.sync_copy(val_v, sh_deg.at[idx_v], add=True)
            return 0

        lax.fori_loop(0, cpw, body, 0)
        plsc.subcore_barrier()
        pltpu.sync_copy(
            sh_deg.at[pl.ds(sid * ZROWS, ZROWS)],
            out_hbm.at[cid, pl.ds(sid * ZROWS, ZROWS)],
        )

    @functools.partial(
        pl.kernel,
        out_type=jax.ShapeDtypeStruct((nchunk, CH), jnp.float32),
        mesh=mesh,
        compiler_params=sc_params,
        scratch_types=[
            pltpu.VMEM((NC, NP), jnp.float32),
            pltpu.VMEM((NP,), jnp.float32),
            pltpu.VMEM((CH,), jnp.int32),
            pltpu.VMEM((CH,), jnp.int32),
            pltpu.VMEM((CH,), jnp.float32),
            pltpu.VMEM((CH,), jnp.float32),
        ],
    )
    def norm_kernel(degp_hbm, row_hbm, col_hbm, w_hbm, out_hbm,
                    dp_v, dis_v, ri, ci, wv, nv):
        cid = lax.axis_index("c")
        sid = lax.axis_index("s")
        pltpu.sync_copy(degp_hbm, dp_v)

        def db(i, _):
            s = pl.ds(i * 16, 16)
            dis_v[s] = _rsqrt_approx(dp_v[0, s] + dp_v[1, s])
            return 0

        lax.fori_loop(0, NP // 16, db, 0)

        base = cid * (nchunk // NC) + sid * cpw

        def body(k, _):
            pltpu.sync_copy(row_hbm.at[base + k], ri)
            pltpu.sync_copy(col_hbm.at[base + k], ci)
            pltpu.sync_copy(w_hbm.at[base + k], wv)
            for q in range(CH // 16):
                s = pl.ds(q * 16, 16)
                a = plsc.load_gather(dis_v, [ri[s]])
                b = plsc.load_gather(dis_v, [ci[s]])
                nv[s] = a * wv[s] * b
            pltpu.sync_copy(nv, out_hbm.at[base + k])
            return 0

        lax.fori_loop(0, cpw, body, 0)

    @functools.partial(
        pl.kernel,
        out_type=jax.ShapeDtypeStruct((NC, NP, H), jnp.float32),
        mesh=mesh,
        compiler_params=sc_params,
        scratch_types=[
            pltpu.VMEM((CH,), jnp.int32),
            pltpu.VMEM((CH,), jnp.int32),
            pltpu.VMEM((CH,), jnp.float32),
            pltpu.VMEM((CH, H), jnp.float32),
            pltpu.VMEM((64, H), jnp.float32),
            pltpu.VMEM_SHARED((NP, H), jnp.float32),
        ],
    )
    def agg_kernel(x_hbm, row_hbm, col_hbm, norm_hbm, out_hbm,
                   ri, ci, nv, rows, zb, sh_agg):
        cid = lax.axis_index("c")
        sid = lax.axis_index("s")

        def zloop(i, _):
            for q in range(H // 16):
                zb[i, pl.ds(q * 16, 16)] = jnp.zeros((16,), jnp.float32)
            return 0

        lax.fori_loop(0, 64, zloop, 0)

        def zc(i, _):
            pltpu.sync_copy(zb, sh_agg.at[pl.ds(sid * ZROWS + i * 64, 64)])
            return 0

        lax.fori_loop(0, ZROWS // 64, zc, 0)
        plsc.subcore_barrier()

        base = cid * (nchunk // NC) + sid * cpw

        def body(k, _):
            pltpu.sync_copy(row_hbm.at[base + k], ri)
            pltpu.sync_copy(col_hbm.at[base + k], ci)
            pltpu.sync_copy(norm_hbm.at[base + k], nv)
            pltpu.sync_copy(x_hbm.at[ri], rows)

            def scale(jb, _):
                nvec = nv[pl.ds(jb * 16, 16)]
                for l in range(16):
                    s_ = nvec[l]
                    r = jb * 16 + l
                    for q in range(H // 16):
                        sl = pl.ds(q * 16, 16)
                        rows[r, sl] = rows[r, sl] * s_
                return 0

            lax.fori_loop(0, CH // 16, scale, 0)
            pltpu.sync_copy(rows, sh_agg.at[ci], add=True)
            return 0

        lax.fori_loop(0, cpw, body, 0)
        plsc.subcore_barrier()
        pltpu.sync_copy(
            sh_agg.at[pl.ds(sid * ZROWS, ZROWS)],
            out_hbm.at[cid, pl.ds(sid * ZROWS, ZROWS)],
        )

    return deg_kernel, norm_kernel, agg_kernel


def _gcn_mm(aggp, x0, w, alpha):
    def body(p_ref, x0_ref, w_ref, o_ref):
        agg = p_ref[0, :N, :] + p_ref[1, :N, :]
        h = (1.0 - alpha) * agg + alpha * x0_ref[...]
        o_ref[...] = jnp.maximum(
            jnp.dot(h, w_ref[...], preferred_element_type=jnp.float32), 0.0
        )

    return pl.pallas_call(
        body,
        out_shape=jax.ShapeDtypeStruct((N, H), jnp.float32),
    )(aggp, x0, w)


def _pool(h, batch2d, gate_w, gate_b2, nn_w, nn_b2):
    def body(h_ref, b_ref, gw_ref, gb_ref, nw_ref, nb_ref, o_ref):
        h_ = h_ref[...]
        gT = lax.dot_general(
            gw_ref[...], h_, (((0,), (1,)), ((), ())),
            preferred_element_type=jnp.float32,
        ) + gb_ref[0, 0]
        v = jnp.dot(h_, nw_ref[...], preferred_element_type=jnp.float32)
        v = v + nb_ref[...]
        bat = b_ref[...]
        seg = lax.broadcasted_iota(jnp.int32, (G, N), 0)
        mask = seg == bat
        gbig = jnp.broadcast_to(gT, (G, N))
        m = jnp.max(jnp.where(mask, gbig, -1e30), axis=1, keepdims=True)
        m_n = jnp.sum(jnp.where(mask, jnp.broadcast_to(m, (G, N)), 0.0),
                      axis=0, keepdims=True)
        g = jnp.exp(gT - m_n)
        s = jnp.sum(jnp.where(mask, jnp.broadcast_to(g, (G, N)), 0.0),
                    axis=1, keepdims=True)
        s_n = jnp.sum(jnp.where(mask, jnp.broadcast_to(s, (G, N)), 0.0),
                      axis=0, keepdims=True)
        wn = g / (s_n + 1e-16)
        wm = jnp.where(mask, jnp.broadcast_to(wn, (G, N)), 0.0)
        o_ref[...] = lax.dot_general(
            wm, v, (((1,), (0,)), ((), ())),
            preferred_element_type=jnp.float32,
        )

    return pl.pallas_call(
        body,
        out_shape=jax.ShapeDtypeStruct((G, H), jnp.float32),
    )(h, batch2d, gate_w, gate_b2, nn_w, nn_b2)


def kernel(x, edge_index, edge_attr, batch, w1, w2, w3, gate_w, gate_b, nn_w, nn_b):
    n = x.shape[0]
    e = edge_index.shape[1]
    et = e + n
    cpw = -(-et // (NW * CH))
    nchunk = NW * cpw
    ep = nchunk * CH

    loop = jnp.arange(n, dtype=jnp.int32)
    pad = jnp.zeros((ep - et,), jnp.int32)
    row = jnp.concatenate([edge_index[0], loop, pad]).reshape(nchunk, CH)
    col = jnp.concatenate([edge_index[1], loop, pad]).reshape(nchunk, CH)
    w = jnp.concatenate(
        [edge_attr, jnp.ones((n,), jnp.float32),
         jnp.zeros((ep - et,), jnp.float32)]
    ).reshape(nchunk, CH)

    deg_kernel, norm_kernel, agg_kernel = _make_sc_kernels(nchunk, cpw)

    degp = deg_kernel(col, w, )
    norm = norm_kernel(degp, row, col, w)

    aggp1 = agg_kernel(x, row, col, norm)
    h1 = _gcn_mm(aggp1, x, w1, 0.2)
    aggp2 = agg_kernel(h1, row, col, norm)
    h2 = _gcn_mm(aggp2, x, w2, 0.2)
    aggp3 = agg_kernel(h2, row, col, norm)
    h3 = _gcn_mm(aggp3, x, w3, 0.4)

    batch2d = batch.reshape(1, n).astype(jnp.int32)
    gate_b2 = gate_b.reshape(1, 1)
    nn_b2 = nn_b.reshape(1, H)
    return _pool(h3, batch2d, gate_w, gate_b2, nn_w, nn_b2)


# speedup vs baseline: 14.1678x; 1.8707x over previous
import functools

import jax
import jax.numpy as jnp
from jax import lax
from jax.experimental import pallas as pl
from jax.experimental.pallas import tpu as pltpu
from jax.experimental.pallas import tpu_sc as plsc

N = 10000
H = 128
G = 64
CH = 128
NC = 2
NS = 16
NW = NC * NS
NP = 10240
ZROWS = NP // NS
NB = 3


def _rsqrt_approx(d):
    ib = lax.bitcast_convert_type(d, jnp.int32)
    y = lax.bitcast_convert_type(jnp.int32(0x5F3759DF) - (ib >> 1), jnp.float32)
    hd = 0.5 * d
    y = y * (1.5 - hd * y * y)
    y = y * (1.5 - hd * y * y)
    y = y * (1.5 - hd * y * y)
    return y


def _make_sc_kernels(nchunk, cpw):
    mesh = plsc.VectorSubcoreMesh(core_axis_name="c", subcore_axis_name="s")
    sc_params = pltpu.CompilerParams(needs_layout_passes=False)

    @functools.partial(
        pl.kernel,
        out_type=jax.ShapeDtypeStruct((NC, NP), jnp.float32),
        mesh=mesh,
        compiler_params=sc_params,
        scratch_types=[
            pltpu.VMEM((cpw * CH,), jnp.float32),
            [pltpu.VMEM((CH,), jnp.int32)] * NB,
            pltpu.VMEM((ZROWS,), jnp.float32),
            pltpu.VMEM_SHARED((NP,), jnp.float32),
            pltpu.SemaphoreType.DMA,
            [pltpu.SemaphoreType.DMA] * NB,
            [pltpu.SemaphoreType.DMA] * NB,
        ],
    )
    def deg_kernel(col_hbm, wf_hbm, out_hbm, w_v, colb, zbuf, sh_deg,
                   psem, csem, ssem):
        cid = lax.axis_index("c")
        sid = lax.axis_index("s")
        base = cid * (nchunk // NC) + sid * cpw

        wp = pltpu.async_copy(
            wf_hbm.at[pl.ds(base * CH, cpw * CH)], w_v, psem)

        def zb(i, _):
            zbuf[pl.ds(i * 16, 16)] = jnp.zeros((16,), jnp.float32)
            return 0

        lax.fori_loop(0, ZROWS // 16, zb, 0)
        pltpu.sync_copy(zbuf, sh_deg.at[pl.ds(sid * ZROWS, ZROWS)])
        wp.wait()
        plsc.subcore_barrier()

        def group(g, _):
            for b in range(NB):
                k = g * NB + b

                @pl.when(g > 0)
                def _():
                    pltpu.make_async_copy(
                        w_v.at[pl.ds(0, CH)], sh_deg.at[colb[b]], ssem[b]
                    ).wait()

                coff = pl.multiple_of((base + k) * CH, CH)
                pltpu.async_copy(
                    col_hbm.at[pl.ds(coff, CH)], colb[b], csem[b])
            for b in range(NB):
                k = g * NB + b
                pltpu.make_async_copy(
                    col_hbm.at[pl.ds(0, CH)], colb[b], csem[b]).wait()
                off = pl.multiple_of(k * CH, CH)
                pltpu.async_copy(
                    w_v.at[pl.ds(off, CH)], sh_deg.at[colb[b]], ssem[b],
                    add=True)
            return 0

        lax.fori_loop(0, cpw // NB, group, 0)
        for b in range(NB):
            pltpu.make_async_copy(
                w_v.at[pl.ds(0, CH)], sh_deg.at[colb[b]], ssem[b]).wait()
        plsc.subcore_barrier()
        pltpu.sync_copy(
            sh_deg.at[pl.ds(sid * ZROWS, ZROWS)],
            out_hbm.at[cid, pl.ds(sid * ZROWS, ZROWS)],
        )

    @functools.partial(
        pl.kernel,
        out_type=jax.ShapeDtypeStruct((nchunk * CH,), jnp.float32),
        mesh=mesh,
        compiler_params=sc_params,
        scratch_types=[
            pltpu.VMEM((NC, NP), jnp.float32),
            pltpu.VMEM((NP,), jnp.float32),
            pltpu.VMEM((cpw * CH,), jnp.int32),
            pltpu.VMEM((cpw * CH,), jnp.int32),
            pltpu.VMEM((cpw * CH,), jnp.float32),
            pltpu.VMEM((cpw * CH,), jnp.float32),
            pltpu.SemaphoreType.DMA,
        ],
    )
    def norm_kernel(degp_hbm, row_hbm, col_hbm, w_hbm, out_hbm,
                    dp_v, dis_v, row_v, col_v, w_v, nv, psem):
        cid = lax.axis_index("c")
        sid = lax.axis_index("s")
        base = cid * (nchunk // NC) + sid * cpw
        fb = pl.multiple_of(base * CH, CH)

        p0 = pltpu.async_copy(degp_hbm, dp_v, psem)
        p1 = pltpu.async_copy(row_hbm.at[pl.ds(fb, cpw * CH)], row_v, psem)
        p2 = pltpu.async_copy(col_hbm.at[pl.ds(fb, cpw * CH)], col_v, psem)
        p3 = pltpu.async_copy(w_hbm.at[pl.ds(fb, cpw * CH)], w_v, psem)
        p0.wait()

        def db(i, _):
            s = pl.ds(i * 16, 16)
            dis_v[s] = _rsqrt_approx(dp_v[0, s] + dp_v[1, s])
            return 0

        lax.fori_loop(0, NP // 16, db, 0)
        p1.wait()
        p2.wait()
        p3.wait()

        def body(k, _):
            for q in range(CH // 16):
                s = pl.ds(pl.multiple_of(k * CH + q * 16, 16), 16)
                a = plsc.load_gather(dis_v, [row_v[s]])
                b = plsc.load_gather(dis_v, [col_v[s]])
                nv[s] = a * w_v[s] * b
            return 0

        lax.fori_loop(0, cpw, body, 0)
        pltpu.sync_copy(nv, out_hbm.at[pl.ds(fb, cpw * CH)])

    NA = 2

    @functools.partial(
        pl.kernel,
        out_type=jax.ShapeDtypeStruct((NC, NP, H), jnp.float32),
        mesh=mesh,
        compiler_params=sc_params,
        scratch_types=[
            pltpu.VMEM((cpw * CH,), jnp.int32),
            [pltpu.VMEM((CH,), jnp.int32)] * NA,
            [pltpu.VMEM((CH,), jnp.float32)] * NA,
            [pltpu.VMEM((CH, H), jnp.float32)] * NA,
            pltpu.VMEM((16, H), jnp.float32),
            pltpu.VMEM_SHARED((NP, H), jnp.float32),
            pltpu.SemaphoreType.DMA,
            [pltpu.SemaphoreType.DMA] * NA,
            [pltpu.SemaphoreType.DMA] * NA,
            [pltpu.SemaphoreType.DMA] * NA,
            [pltpu.SemaphoreType.DMA] * NA,
        ],
    )
    def agg_kernel(x_hbm, rowf_hbm, col_hbm, norm_hbm, out_hbm,
                   row_v, colb, normb, rows, zb, sh_agg,
                   psem, csem, nsem, gsem, ssem):
        cid = lax.axis_index("c")
        sid = lax.axis_index("s")
        base = cid * (nchunk // NC) + sid * cpw

        fb = pl.multiple_of(base * CH, CH)
        p1 = pltpu.async_copy(rowf_hbm.at[pl.ds(fb, cpw * CH)], row_v, psem)

        def zloop(i, _):
            for q in range(H // 16):
                zb[i, pl.ds(q * 16, 16)] = jnp.zeros((16,), jnp.float32)
            return 0

        lax.fori_loop(0, 16, zloop, 0)

        def zc(i, _):
            pltpu.sync_copy(zb, sh_agg.at[pl.ds(sid * ZROWS + i * 16, 16)])
            return 0

        lax.fori_loop(0, ZROWS // 16, zc, 0)
        p1.wait()
        plsc.subcore_barrier()

        def group(g, _):
            for b in range(NA):
                k = g * NA + b

                @pl.when(g > 0)
                def _():
                    pltpu.make_async_copy(
                        rows[b], sh_agg.at[colb[b]], ssem[b]).wait()

                coff = pl.multiple_of((base + k) * CH, CH)
                pltpu.async_copy(
                    col_hbm.at[pl.ds(coff, CH)], colb[b], csem[b])
                pltpu.async_copy(
                    norm_hbm.at[pl.ds(coff, CH)], normb[b], nsem[b])
                off = pl.multiple_of(k * CH, CH)
                pltpu.async_copy(
                    x_hbm.at[row_v.at[pl.ds(off, CH)]], rows[b], gsem[b])
            for b in range(NA):
                pltpu.make_async_copy(
                    x_hbm.at[row_v.at[pl.ds(0, CH)]], rows[b], gsem[b]
                ).wait()
                pltpu.make_async_copy(
                    norm_hbm.at[pl.ds(0, CH)], normb[b], nsem[b]).wait()

                def scale(jb, _):
                    nvec = normb[b][pl.ds(jb * 16, 16)]
                    for l in range(16):
                        s_ = nvec[l]
                        r = jb * 16 + l
                        for q in range(H // 16):
                            sl = pl.ds(q * 16, 16)
                            rows[b][r, sl] = rows[b][r, sl] * s_
                    return 0

                lax.fori_loop(0, CH // 16, scale, 0)
                pltpu.make_async_copy(
                    col_hbm.at[pl.ds(0, CH)], colb[b], csem[b]).wait()
                pltpu.async_copy(rows[b], sh_agg.at[colb[b]], ssem[b], add=True)
            return 0

        lax.fori_loop(0, cpw // NA, group, 0)

        if cpw % NA:
            for b in range(cpw % NA):
                k = (cpw // NA) * NA + b
                pltpu.make_async_copy(
                    rows[b], sh_agg.at[colb[b]], ssem[b]).wait()
                coff = pl.multiple_of((base + k) * CH, CH)
                pltpu.async_copy(
                    col_hbm.at[pl.ds(coff, CH)], colb[b], csem[b])
                pltpu.async_copy(
                    norm_hbm.at[pl.ds(coff, CH)], normb[b], nsem[b])
                off = pl.multiple_of(k * CH, CH)
                pltpu.async_copy(
                    x_hbm.at[row_v.at[pl.ds(off, CH)]], rows[b], gsem[b])
            for b in range(cpw % NA):
                pltpu.make_async_copy(
                    x_hbm.at[row_v.at[pl.ds(0, CH)]], rows[b], gsem[b]
                ).wait()
                pltpu.make_async_copy(
                    norm_hbm.at[pl.ds(0, CH)], normb[b], nsem[b]).wait()

                def tscale(jb, _):
                    nvec = normb[b][pl.ds(jb * 16, 16)]
                    for l in range(16):
                        s_ = nvec[l]
                        r = jb * 16 + l
                        for q in range(H // 16):
                            sl = pl.ds(q * 16, 16)
                            rows[b][r, sl] = rows[b][r, sl] * s_
                    return 0

                lax.fori_loop(0, CH // 16, tscale, 0)
                pltpu.make_async_copy(
                    col_hbm.at[pl.ds(0, CH)], colb[b], csem[b]).wait()
                pltpu.async_copy(rows[b], sh_agg.at[colb[b]], ssem[b], add=True)

        for b in range(NA):
            pltpu.make_async_copy(rows[b], sh_agg.at[colb[b]], ssem[b]).wait()
        plsc.subcore_barrier()
        pltpu.sync_copy(
            sh_agg.at[pl.ds(sid * ZROWS, ZROWS)],
            out_hbm.at[cid, pl.ds(sid * ZROWS, ZROWS)],
        )

    return deg_kernel, norm_kernel, agg_kernel


def _gcn_mm(aggp, x0, w, alpha):
    def body(p_ref, x0_ref, w_ref, o_ref):
        agg = p_ref[0, :N, :] + p_ref[1, :N, :]
        h = (1.0 - alpha) * agg + alpha * x0_ref[...]
        o_ref[...] = jnp.maximum(
            jnp.dot(h, w_ref[...], preferred_element_type=jnp.float32), 0.0
        )

    return pl.pallas_call(
        body,
        out_shape=jax.ShapeDtypeStruct((N, H), jnp.float32),
    )(aggp, x0, w)


def _pool(h, batch2d, gate_w, gate_b2, nn_w, nn_b2):
    def body(h_ref, b_ref, gw_ref, gb_ref, nw_ref, nb_ref, o_ref):
        h_ = h_ref[...]
        gT = lax.dot_general(
            gw_ref[...], h_, (((0,), (1,)), ((), ())),
            preferred_element_type=jnp.float32,
        ) + gb_ref[0, 0]
        v = jnp.dot(h_, nw_ref[...], preferred_element_type=jnp.float32)
        v = v + nb_ref[...]
        bat = b_ref[...]
        seg = lax.broadcasted_iota(jnp.int32, (G, N), 0)
        mask = seg == bat
        gbig = jnp.broadcast_to(gT, (G, N))
        m = jnp.max(jnp.where(mask, gbig, -1e30), axis=1, keepdims=True)
        m_n = jnp.sum(jnp.where(mask, jnp.broadcast_to(m, (G, N)), 0.0),
                      axis=0, keepdims=True)
        g = jnp.exp(gT - m_n)
        s = jnp.sum(jnp.where(mask, jnp.broadcast_to(g, (G, N)), 0.0),
                    axis=1, keepdims=True)
        s_n = jnp.sum(jnp.where(mask, jnp.broadcast_to(s, (G, N)), 0.0),
                      axis=0, keepdims=True)
        wn = g / (s_n + 1e-16)
        wm = jnp.where(mask, jnp.broadcast_to(wn, (G, N)), 0.0)
        o_ref[...] = lax.dot_general(
            wm, v, (((1,), (0,)), ((), ())),
            preferred_element_type=jnp.float32,
        )

    return pl.pallas_call(
        body,
        out_shape=jax.ShapeDtypeStruct((G, H), jnp.float32),
    )(h, batch2d, gate_w, gate_b2, nn_w, nn_b2)


def kernel(x, edge_index, edge_attr, batch, w1, w2, w3, gate_w, gate_b, nn_w, nn_b):
    n = x.shape[0]
    e = edge_index.shape[1]
    et = e + n
    cpw = -(-et // (NW * CH * NB)) * NB
    nchunk = NW * cpw
    ep = nchunk * CH

    loop = jnp.arange(n, dtype=jnp.int32)
    pad = jnp.zeros((ep - et,), jnp.int32)
    rowf = jnp.concatenate([edge_index[0], loop, pad])
    colf = jnp.concatenate([edge_index[1], loop, pad])
    wf = jnp.concatenate(
        [edge_attr, jnp.ones((n,), jnp.float32),
         jnp.zeros((ep - et,), jnp.float32)]
    )

    deg_kernel, norm_kernel, agg_kernel = _make_sc_kernels(nchunk, cpw)

    degp = deg_kernel(colf, wf)
    normf = norm_kernel(degp, rowf, colf, wf)

    aggp1 = agg_kernel(x, rowf, colf, normf)
    h1 = _gcn_mm(aggp1, x, w1, 0.2)
    aggp2 = agg_kernel(h1, rowf, colf, normf)
    h2 = _gcn_mm(aggp2, x, w2, 0.2)
    aggp3 = agg_kernel(h2, rowf, colf, normf)
    h3 = _gcn_mm(aggp3, x, w3, 0.4)

    batch2d = batch.reshape(1, n).astype(jnp.int32)
    gate_b2 = gate_b.reshape(1, 1)
    nn_b2 = nn_b.reshape(1, H)
    return _pool(h3, batch2d, gate_w, gate_b2, nn_w, nn_b2)


# speedup vs baseline: 18.0122x; 1.2713x over previous
import functools

import jax
import jax.numpy as jnp
from jax import lax
from jax.experimental import pallas as pl
from jax.experimental.pallas import tpu as pltpu
from jax.experimental.pallas import tpu_sc as plsc

N = 10000
H = 128
G = 64
CH = 128
NC = 2
NS = 16
NW = NC * NS
NP = 10240
ZROWS = NP // NS
NB = 3


def _rsqrt_approx(d):
    ib = lax.bitcast_convert_type(d, jnp.int32)
    y = lax.bitcast_convert_type(jnp.int32(0x5F3759DF) - (ib >> 1), jnp.float32)
    hd = 0.5 * d
    y = y * (1.5 - hd * y * y)
    y = y * (1.5 - hd * y * y)
    y = y * (1.5 - hd * y * y)
    return y


def _make_sc_kernels(nchunk, cpw):
    mesh = plsc.VectorSubcoreMesh(core_axis_name="c", subcore_axis_name="s")
    sc_params = pltpu.CompilerParams(needs_layout_passes=False)

    @functools.partial(
        pl.kernel,
        out_type=jax.ShapeDtypeStruct((NC, NP), jnp.float32),
        mesh=mesh,
        compiler_params=sc_params,
        scratch_types=[
            pltpu.VMEM((cpw * CH,), jnp.float32),
            [pltpu.VMEM((CH,), jnp.int32)] * NB,
            pltpu.VMEM((ZROWS,), jnp.float32),
            pltpu.VMEM_SHARED((NP,), jnp.float32),
            pltpu.SemaphoreType.DMA,
            [pltpu.SemaphoreType.DMA] * NB,
            [pltpu.SemaphoreType.DMA] * NB,
        ],
    )
    def deg_kernel(col_hbm, wf_hbm, out_hbm, w_v, colb, zbuf, sh_deg,
                   psem, csem, ssem):
        cid = lax.axis_index("c")
        sid = lax.axis_index("s")
        base = cid * (nchunk // NC) + sid * cpw

        wp = pltpu.async_copy(
            wf_hbm.at[pl.ds(base * CH, cpw * CH)], w_v, psem)

        def zb(i, _):
            zbuf[pl.ds(i * 16, 16)] = jnp.zeros((16,), jnp.float32)
            return 0

        lax.fori_loop(0, ZROWS // 16, zb, 0)
        pltpu.sync_copy(zbuf, sh_deg.at[pl.ds(sid * ZROWS, ZROWS)])
        wp.wait()
        plsc.subcore_barrier()

        def group(g, _):
            for b in range(NB):
                k = g * NB + b

                @pl.when(g > 0)
                def _():
                    pltpu.make_async_copy(
                        w_v.at[pl.ds(0, CH)], sh_deg.at[colb[b]], ssem[b]
                    ).wait()

                coff = pl.multiple_of((base + k) * CH, CH)
                pltpu.async_copy(
                    col_hbm.at[pl.ds(coff, CH)], colb[b], csem[b])
            for b in range(NB):
                k = g * NB + b
                pltpu.make_async_copy(
                    col_hbm.at[pl.ds(0, CH)], colb[b], csem[b]).wait()
                off = pl.multiple_of(k * CH, CH)
                pltpu.async_copy(
                    w_v.at[pl.ds(off, CH)], sh_deg.at[colb[b]], ssem[b],
                    add=True)
            return 0

        lax.fori_loop(0, cpw // NB, group, 0)
        for b in range(NB):
            pltpu.make_async_copy(
                w_v.at[pl.ds(0, CH)], sh_deg.at[colb[b]], ssem[b]).wait()
        plsc.subcore_barrier()
        pltpu.sync_copy(
            sh_deg.at[pl.ds(sid * ZROWS, ZROWS)],
            out_hbm.at[cid, pl.ds(sid * ZROWS, ZROWS)],
        )

    @functools.partial(
        pl.kernel,
        out_type=jax.ShapeDtypeStruct((nchunk * CH,), jnp.float32),
        mesh=mesh,
        compiler_params=sc_params,
        scratch_types=[
            pltpu.VMEM((NC, NP), jnp.float32),
            pltpu.VMEM((NP,), jnp.float32),
            pltpu.VMEM((cpw * CH,), jnp.int32),
            pltpu.VMEM((cpw * CH,), jnp.int32),
            pltpu.VMEM((cpw * CH,), jnp.float32),
            pltpu.VMEM((cpw * CH,), jnp.float32),
            pltpu.SemaphoreType.DMA,
        ],
    )
    def norm_kernel(degp_hbm, row_hbm, col_hbm, w_hbm, out_hbm,
                    dp_v, dis_v, row_v, col_v, w_v, nv, psem):
        cid = lax.axis_index("c")
        sid = lax.axis_index("s")
        base = cid * (nchunk // NC) + sid * cpw
        fb = pl.multiple_of(base * CH, CH)

        p0 = pltpu.async_copy(degp_hbm, dp_v, psem)
        p1 = pltpu.async_copy(row_hbm.at[pl.ds(fb, cpw * CH)], row_v, psem)
        p2 = pltpu.async_copy(col_hbm.at[pl.ds(fb, cpw * CH)], col_v, psem)
        p3 = pltpu.async_copy(w_hbm.at[pl.ds(fb, cpw * CH)], w_v, psem)
        p0.wait()

        def db(i, _):
            s = pl.ds(i * 16, 16)
            dis_v[s] = _rsqrt_approx(dp_v[0, s] + dp_v[1, s])
            return 0

        lax.fori_loop(0, NP // 16, db, 0)
        p1.wait()
        p2.wait()
        p3.wait()

        def body(k, _):
            for q in range(CH // 16):
                s = pl.ds(pl.multiple_of(k * CH + q * 16, 16), 16)
                a = plsc.load_gather(dis_v, [row_v[s]])
                b = plsc.load_gather(dis_v, [col_v[s]])
                nv[s] = a * w_v[s] * b
            return 0

        lax.fori_loop(0, cpw, body, 0)
        pltpu.sync_copy(nv, out_hbm.at[pl.ds(fb, cpw * CH)])

    NA = 2

    @functools.partial(
        pl.kernel,
        out_type=jax.ShapeDtypeStruct((NC, NP, H), jnp.float32),
        mesh=mesh,
        compiler_params=sc_params,
        scratch_types=[
            [pltpu.VMEM((CH,), jnp.int32)] * NA,
            [pltpu.VMEM((CH,), jnp.int32)] * NA,
            [pltpu.VMEM((CH,), jnp.float32)] * NA,
            [pltpu.VMEM((CH, H), jnp.float32)] * NA,
            pltpu.VMEM((16, H), jnp.float32),
            pltpu.VMEM_SHARED((NP, H), jnp.float32),
            [pltpu.SemaphoreType.DMA] * NA,
            [pltpu.SemaphoreType.DMA] * NA,
            [pltpu.SemaphoreType.DMA] * NA,
            [pltpu.SemaphoreType.DMA] * NA,
            [pltpu.SemaphoreType.DMA] * NA,
        ],
    )
    def agg_kernel(x_hbm, rowf_hbm, col_hbm, norm_hbm, out_hbm,
                   rowb, colb, normb, rowsb, zb, sh_agg,
                   rsem, csem, nsem, gsem, ssem):
        cid = lax.axis_index("c")
        sid = lax.axis_index("s")
        base = cid * (nchunk // NC) + sid * cpw

        def zloop(i, _):
            for q in range(H // 16):
                zb[i, pl.ds(q * 16, 16)] = jnp.zeros((16,), jnp.float32)
            return 0

        lax.fori_loop(0, 16, zloop, 0)

        def zc(i, _):
            pltpu.sync_copy(zb, sh_agg.at[pl.ds(sid * ZROWS + i * 16, 16)])
            return 0

        lax.fori_loop(0, ZROWS // 16, zc, 0)
        plsc.subcore_barrier()

        def do_chunks(ks, nb, first):
            for b in range(nb):
                k = ks + b
                if not first:
                    pltpu.make_async_copy(
                        rowsb[b], sh_agg.at[colb[b]], ssem[b]).wait()
                coff = pl.multiple_of((base + k) * CH, CH)
                pltpu.async_copy(
                    rowf_hbm.at[pl.ds(coff, CH)], rowb[b], rsem[b])
                pltpu.async_copy(
                    col_hbm.at[pl.ds(coff, CH)], colb[b], csem[b])
                pltpu.async_copy(
                    norm_hbm.at[pl.ds(coff, CH)], normb[b], nsem[b])
            for b in range(nb):
                pltpu.make_async_copy(
                    rowf_hbm.at[pl.ds(0, CH)], rowb[b], rsem[b]).wait()
                pltpu.async_copy(x_hbm.at[rowb[b]], rowsb[b], gsem[b])
            for b in range(nb):
                pltpu.make_async_copy(
                    x_hbm.at[rowb[b]], rowsb[b], gsem[b]).wait()
                pltpu.make_async_copy(
                    norm_hbm.at[pl.ds(0, CH)], normb[b], nsem[b]).wait()

                def scale(jb, _):
                    nvec = normb[b][pl.ds(jb * 16, 16)]
                    for l in range(16):
                        s_ = nvec[l]
                        r = jb * 16 + l
                        for q in range(H // 16):
                            rowsb[b][r, pl.ds(q * 16, 16)] = (
                                rowsb[b][r, pl.ds(q * 16, 16)] * s_)
                    return 0

                lax.fori_loop(0, CH // 16, scale, 0)
                pltpu.make_async_copy(
                    col_hbm.at[pl.ds(0, CH)], colb[b], csem[b]).wait()
                pltpu.async_copy(
                    rowsb[b], sh_agg.at[colb[b]], ssem[b], add=True)

        def group(g, _):
            do_chunks(g * NA, NA, False)
            return 0

        do_chunks(0, NA, True)
        lax.fori_loop(1, cpw // NA, group, 0)
        if cpw % NA:
            do_chunks((cpw // NA) * NA, cpw % NA, False)

        for b in range(NA):
            pltpu.make_async_copy(rowsb[b], sh_agg.at[colb[b]], ssem[b]).wait()
        plsc.subcore_barrier()
        pltpu.sync_copy(
            sh_agg.at[pl.ds(sid * ZROWS, ZROWS)],
            out_hbm.at[cid, pl.ds(sid * ZROWS, ZROWS)],
        )

    return deg_kernel, norm_kernel, agg_kernel


def _gcn_mm(aggp, x0, w, alpha):
    def body(p_ref, x0_ref, w_ref, o_ref):
        agg = p_ref[0, :N, :] + p_ref[1, :N, :]
        h = (1.0 - alpha) * agg + alpha * x0_ref[...]
        o_ref[...] = jnp.maximum(
            jnp.dot(h, w_ref[...], preferred_element_type=jnp.float32), 0.0
        )

    return pl.pallas_call(
        body,
        out_shape=jax.ShapeDtypeStruct((N, H), jnp.float32),
    )(aggp, x0, w)


def _pool(h, batch2d, gate_w, gate_b2, nn_w, nn_b2):
    def body(h_ref, b_ref, gw_ref, gb_ref, nw_ref, nb_ref, o_ref):
        h_ = h_ref[...]
        gT = lax.dot_general(
            gw_ref[...], h_, (((0,), (1,)), ((), ())),
            preferred_element_type=jnp.float32,
        ) + gb_ref[0, 0]
        v = jnp.dot(h_, nw_ref[...], preferred_element_type=jnp.float32)
        v = v + nb_ref[...]
        bat = b_ref[...]
        seg = lax.broadcasted_iota(jnp.int32, (G, N), 0)
        mask = seg == bat
        gbig = jnp.broadcast_to(gT, (G, N))
        m = jnp.max(jnp.where(mask, gbig, -1e30), axis=1, keepdims=True)
        m_n = jnp.sum(jnp.where(mask, jnp.broadcast_to(m, (G, N)), 0.0),
                      axis=0, keepdims=True)
        g = jnp.exp(gT - m_n)
        s = jnp.sum(jnp.where(mask, jnp.broadcast_to(g, (G, N)), 0.0),
                    axis=1, keepdims=True)
        s_n = jnp.sum(jnp.where(mask, jnp.broadcast_to(s, (G, N)), 0.0),
                      axis=0, keepdims=True)
        wn = g / (s_n + 1e-16)
        wm = jnp.where(mask, jnp.broadcast_to(wn, (G, N)), 0.0)
        o_ref[...] = lax.dot_general(
            wm, v, (((1,), (0,)), ((), ())),
            preferred_element_type=jnp.float32,
        )

    return pl.pallas_call(
        body,
        out_shape=jax.ShapeDtypeStruct((G, H), jnp.float32),
    )(h, batch2d, gate_w, gate_b2, nn_w, nn_b2)


def kernel(x, edge_index, edge_attr, batch, w1, w2, w3, gate_w, gate_b, nn_w, nn_b):
    n = x.shape[0]
    e = edge_index.shape[1]
    et = e + n
    cpw = -(-et // (NW * CH * NB)) * NB
    nchunk = NW * cpw
    ep = nchunk * CH

    loop = jnp.arange(n, dtype=jnp.int32)
    pad = jnp.arange(ep - et, dtype=jnp.int32) % n
    rowf = jnp.concatenate([edge_index[0], loop, pad])
    colf = jnp.concatenate([edge_index[1], loop, pad])
    wf = jnp.concatenate(
        [edge_attr, jnp.ones((n,), jnp.float32),
         jnp.zeros((ep - et,), jnp.float32)]
    )

    deg_kernel, norm_kernel, agg_kernel = _make_sc_kernels(nchunk, cpw)

    degp = deg_kernel(colf, wf)
    normf = norm_kernel(degp, rowf, colf, wf)

    aggp1 = agg_kernel(x, rowf, colf, normf)
    h1 = _gcn_mm(aggp1, x, w1, 0.2)
    aggp2 = agg_kernel(h1, rowf, colf, normf)
    h2 = _gcn_mm(aggp2, x, w2, 0.2)
    aggp3 = agg_kernel(h2, rowf, colf, normf)
    h3 = _gcn_mm(aggp3, x, w3, 0.4)

    batch2d = batch.reshape(1, n).astype(jnp.int32)
    gate_b2 = gate_b.reshape(1, 1)
    nn_b2 = nn_b.reshape(1, H)
    return _pool(h3, batch2d, gate_w, gate_b2, nn_w, nn_b2)


# speedup vs baseline: 18.3343x; 1.0179x over previous
import functools

import jax
import jax.numpy as jnp
from jax import lax
from jax.experimental import pallas as pl
from jax.experimental.pallas import tpu as pltpu
from jax.experimental.pallas import tpu_sc as plsc

N = 10000
H = 128
G = 64
CH = 128
NC = 2
NS = 16
NW = NC * NS
NP = 10240
ZROWS = NP // NS
NB = 3


def _make_sc_kernels(nchunk, cpw):
    mesh = plsc.VectorSubcoreMesh(core_axis_name="c", subcore_axis_name="s")
    sc_params = pltpu.CompilerParams(needs_layout_passes=False)

    @functools.partial(
        pl.kernel,
        out_type=jax.ShapeDtypeStruct((NC, NP), jnp.float32),
        mesh=mesh,
        compiler_params=sc_params,
        scratch_types=[
            pltpu.VMEM((cpw * CH,), jnp.float32),
            [pltpu.VMEM((CH,), jnp.int32)] * NB,
            pltpu.VMEM((ZROWS,), jnp.float32),
            pltpu.VMEM_SHARED((NP,), jnp.float32),
            pltpu.SemaphoreType.DMA,
            [pltpu.SemaphoreType.DMA] * NB,
            [pltpu.SemaphoreType.DMA] * NB,
        ],
    )
    def deg_kernel(col_hbm, wf_hbm, out_hbm, w_v, colb, zbuf, sh_deg,
                   psem, csem, ssem):
        cid = lax.axis_index("c")
        sid = lax.axis_index("s")
        base = cid * (nchunk // NC) + sid * cpw

        wp = pltpu.async_copy(
            wf_hbm.at[pl.ds(base * CH, cpw * CH)], w_v, psem)

        def zb(i, _):
            zbuf[pl.ds(i * 16, 16)] = jnp.zeros((16,), jnp.float32)
            return 0

        lax.fori_loop(0, ZROWS // 16, zb, 0)
        pltpu.sync_copy(zbuf, sh_deg.at[pl.ds(sid * ZROWS, ZROWS)])
        wp.wait()
        plsc.subcore_barrier()

        def group(g, _):
            for b in range(NB):
                k = g * NB + b

                @pl.when(g > 0)
                def _():
                    pltpu.make_async_copy(
                        w_v.at[pl.ds(0, CH)], sh_deg.at[colb[b]], ssem[b]
                    ).wait()

                coff = pl.multiple_of((base + k) * CH, CH)
                pltpu.async_copy(
                    col_hbm.at[pl.ds(coff, CH)], colb[b], csem[b])
            for b in range(NB):
                k = g * NB + b
                pltpu.make_async_copy(
                    col_hbm.at[pl.ds(0, CH)], colb[b], csem[b]).wait()
                off = pl.multiple_of(k * CH, CH)
                pltpu.async_copy(
                    w_v.at[pl.ds(off, CH)], sh_deg.at[colb[b]], ssem[b],
                    add=True)
            return 0

        lax.fori_loop(0, cpw // NB, group, 0)
        for b in range(NB):
            pltpu.make_async_copy(
                w_v.at[pl.ds(0, CH)], sh_deg.at[colb[b]], ssem[b]).wait()
        plsc.subcore_barrier()
        pltpu.sync_copy(
            sh_deg.at[pl.ds(sid * ZROWS, ZROWS)],
            out_hbm.at[cid, pl.ds(sid * ZROWS, ZROWS)],
        )

    NA = 2

    @functools.partial(
        pl.kernel,
        out_type=jax.ShapeDtypeStruct((NC, NP, H), jnp.float32),
        mesh=mesh,
        compiler_params=sc_params,
        scratch_types=[
            [pltpu.VMEM((CH,), jnp.int32)] * NA,
            [pltpu.VMEM((CH,), jnp.int32)] * NA,
            [pltpu.VMEM((CH,), jnp.float32)] * NA,
            [pltpu.VMEM((CH, H), jnp.float32)] * NA,
            pltpu.VMEM((16, H), jnp.float32),
            pltpu.VMEM_SHARED((NP, H), jnp.float32),
            [pltpu.SemaphoreType.DMA] * NA,
            [pltpu.SemaphoreType.DMA] * NA,
            [pltpu.SemaphoreType.DMA] * NA,
            [pltpu.SemaphoreType.DMA] * NA,
            [pltpu.SemaphoreType.DMA] * NA,
        ],
    )
    def agg_kernel(x_hbm, rowf_hbm, col_hbm, norm_hbm, out_hbm,
                   rowb, colb, normb, rowsb, zb, sh_agg,
                   rsem, csem, nsem, gsem, ssem):
        cid = lax.axis_index("c")
        sid = lax.axis_index("s")
        base = cid * (nchunk // NC) + sid * cpw

        def zloop(i, _):
            for q in range(H // 16):
                zb[i, pl.ds(q * 16, 16)] = jnp.zeros((16,), jnp.float32)
            return 0

        lax.fori_loop(0, 16, zloop, 0)

        def zc(i, _):
            pltpu.sync_copy(zb, sh_agg.at[pl.ds(sid * ZROWS + i * 16, 16)])
            return 0

        lax.fori_loop(0, ZROWS // 16, zc, 0)
        plsc.subcore_barrier()

        def do_chunks(ks, nb, first):
            for b in range(nb):
                k = ks + b
                if not first:
                    pltpu.make_async_copy(
                        rowsb[b], sh_agg.at[colb[b]], ssem[b]).wait()
                coff = pl.multiple_of((base + k) * CH, CH)
                pltpu.async_copy(
                    rowf_hbm.at[pl.ds(coff, CH)], rowb[b], rsem[b])
                pltpu.async_copy(
                    col_hbm.at[pl.ds(coff, CH)], colb[b], csem[b])
                pltpu.async_copy(
                    norm_hbm.at[pl.ds(coff, CH)], normb[b], nsem[b])
            for b in range(nb):
                pltpu.make_async_copy(
                    rowf_hbm.at[pl.ds(0, CH)], rowb[b], rsem[b]).wait()
                pltpu.async_copy(x_hbm.at[rowb[b]], rowsb[b], gsem[b])
            for b in range(nb):
                pltpu.make_async_copy(
                    x_hbm.at[rowb[b]], rowsb[b], gsem[b]).wait()
                pltpu.make_async_copy(
                    norm_hbm.at[pl.ds(0, CH)], normb[b], nsem[b]).wait()

                def scale(jb, _):
                    nvec = normb[b][pl.ds(jb * 16, 16)]
                    for l in range(16):
                        s_ = nvec[l]
                        r = jb * 16 + l
                        for q in range(H // 16):
                            rowsb[b][r, pl.ds(q * 16, 16)] = (
                                rowsb[b][r, pl.ds(q * 16, 16)] * s_)
                    return 0

                lax.fori_loop(0, CH // 16, scale, 0)
                pltpu.make_async_copy(
                    col_hbm.at[pl.ds(0, CH)], colb[b], csem[b]).wait()
                pltpu.async_copy(
                    rowsb[b], sh_agg.at[colb[b]], ssem[b], add=True)

        def group(g, _):
            do_chunks(g * NA, NA, False)
            return 0

        do_chunks(0, NA, True)
        lax.fori_loop(1, cpw // NA, group, 0)
        if cpw % NA:
            do_chunks((cpw // NA) * NA, cpw % NA, False)

        for b in range(NA):
            pltpu.make_async_copy(rowsb[b], sh_agg.at[colb[b]], ssem[b]).wait()
        plsc.subcore_barrier()
        pltpu.sync_copy(
            sh_agg.at[pl.ds(sid * ZROWS, ZROWS)],
            out_hbm.at[cid, pl.ds(sid * ZROWS, ZROWS)],
        )

    return deg_kernel, agg_kernel


def _prep(degpT, x):
    def body(p_ref, x_ref, dis_ref, xs_ref):
        d = p_ref[...]
        dis = lax.rsqrt(d[:, 0:1] + d[:, 1:2])
        dis_ref[...] = dis
        xs_ref[...] = x_ref[...] * dis[:N]

    return pl.pallas_call(
        body,
        out_shape=(
            jax.ShapeDtypeStruct((NP, 1), jnp.float32),
            jax.ShapeDtypeStruct((N, H), jnp.float32),
        ),
    )(degpT, x)


def _gcn_mm(aggp, dis, x0, w, alpha, scale_out):
    def body(p_ref, d_ref, x0_ref, w_ref, o_ref):
        d = d_ref[...][:N]
        agg = (p_ref[0, :N, :] + p_ref[1, :N, :]) * d
        h = (1.0 - alpha) * agg + alpha * x0_ref[...]
        o = jnp.maximum(
            jnp.dot(h, w_ref[...], preferred_element_type=jnp.float32), 0.0
        )
        if scale_out:
            o = o * d
        o_ref[...] = o

    return pl.pallas_call(
        body,
        out_shape=jax.ShapeDtypeStruct((N, H), jnp.float32),
    )(aggp, dis, x0, w)


def _pool(h, batch2d, gate_w, gate_b2, nn_w, nn_b2):
    def body(h_ref, b_ref, gw_ref, gb_ref, nw_ref, nb_ref, o_ref):
        h_ = h_ref[...]
        gT = lax.dot_general(
            gw_ref[...], h_, (((0,), (1,)), ((), ())),
            preferred_element_type=jnp.float32,
        ) + gb_ref[0, 0]
        v = jnp.dot(h_, nw_ref[...], preferred_element_type=jnp.float32)
        v = v + nb_ref[...]
        bat = b_ref[...]
        seg = lax.broadcasted_iota(jnp.int32, (G, N), 0)
        mask = seg == bat
        gbig = jnp.broadcast_to(gT, (G, N))
        m = jnp.max(jnp.where(mask, gbig, -1e30), axis=1, keepdims=True)
        m_n = jnp.sum(jnp.where(mask, jnp.broadcast_to(m, (G, N)), 0.0),
                      axis=0, keepdims=True)
        g = jnp.exp(gT - m_n)
        s = jnp.sum(jnp.where(mask, jnp.broadcast_to(g, (G, N)), 0.0),
                    axis=1, keepdims=True)
        s_n = jnp.sum(jnp.where(mask, jnp.broadcast_to(s, (G, N)), 0.0),
                      axis=0, keepdims=True)
        wn = g / (s_n + 1e-16)
        wm = jnp.where(mask, jnp.broadcast_to(wn, (G, N)), 0.0)
        o_ref[...] = lax.dot_general(
            wm, v, (((1,), (0,)), ((), ())),
            preferred_element_type=jnp.float32,
        )

    return pl.pallas_call(
        body,
        out_shape=jax.ShapeDtypeStruct((G, H), jnp.float32),
    )(h, batch2d, gate_w, gate_b2, nn_w, nn_b2)


def kernel(x, edge_index, edge_attr, batch, w1, w2, w3, gate_w, gate_b, nn_w, nn_b):
    n = x.shape[0]
    e = edge_index.shape[1]
    et = e + n
    cpw = -(-et // (NW * CH * NB)) * NB
    nchunk = NW * cpw
    ep = nchunk * CH

    loop = jnp.arange(n, dtype=jnp.int32)
    pad = jnp.arange(ep - et, dtype=jnp.int32) % n
    rowf = jnp.concatenate([edge_index[0], loop, pad])
    colf = jnp.concatenate([edge_index[1], loop, pad])
    wf = jnp.concatenate(
        [edge_attr, jnp.ones((n,), jnp.float32),
         jnp.zeros((ep - et,), jnp.float32)]
    )

    deg_kernel, agg_kernel = _make_sc_kernels(nchunk, cpw)

    degp = deg_kernel(colf, wf)
    dis, xs = _prep(degp.T, x)

    aggp1 = agg_kernel(xs, rowf, colf, wf)
    h1 = _gcn_mm(aggp1, dis, x, w1, 0.2, True)
    aggp2 = agg_kernel(h1, rowf, colf, wf)
    h2 = _gcn_mm(aggp2, dis, x, w2, 0.2, True)
    aggp3 = agg_kernel(h2, rowf, colf, wf)
    h3 = _gcn_mm(aggp3, dis, x, w3, 0.4, False)

    batch2d = batch.reshape(1, n).astype(jnp.int32)
    gate_b2 = gate_b.reshape(1, 1)
    nn_b2 = nn_b.reshape(1, H)
    return _pool(h3, batch2d, gate_w, gate_b2, nn_w, nn_b2)


# speedup vs baseline: 18.4581x; 1.0067x over previous
import functools

import jax
import jax.numpy as jnp
from jax import lax
from jax.experimental import pallas as pl
from jax.experimental.pallas import tpu as pltpu
from jax.experimental.pallas import tpu_sc as plsc

N = 10000
H = 128
G = 64
CH = 128
NC = 2
NS = 16
NW = NC * NS
NP = 10240
ZROWS = NP // NS
NB = 3


def _make_sc_kernels(nchunk, cpw):
    mesh = plsc.VectorSubcoreMesh(core_axis_name="c", subcore_axis_name="s")
    sc_params = pltpu.CompilerParams(needs_layout_passes=False)

    @functools.partial(
        pl.kernel,
        out_type=jax.ShapeDtypeStruct((NC, NP), jnp.float32),
        mesh=mesh,
        compiler_params=sc_params,
        scratch_types=[
            pltpu.VMEM((cpw * CH,), jnp.float32),
            [pltpu.VMEM((CH,), jnp.int32)] * NB,
            pltpu.VMEM((ZROWS,), jnp.float32),
            pltpu.VMEM_SHARED((NP,), jnp.float32),
            pltpu.SemaphoreType.DMA,
            [pltpu.SemaphoreType.DMA] * NB,
            [pltpu.SemaphoreType.DMA] * NB,
        ],
    )
    def deg_kernel(col_hbm, wf_hbm, out_hbm, w_v, colb, zbuf, sh_deg,
                   psem, csem, ssem):
        cid = lax.axis_index("c")
        sid = lax.axis_index("s")
        base = cid * (nchunk // NC) + sid * cpw

        wp = pltpu.async_copy(
            wf_hbm.at[pl.ds(base * CH, cpw * CH)], w_v, psem)

        def zb(i, _):
            zbuf[pl.ds(i * 16, 16)] = jnp.zeros((16,), jnp.float32)
            return 0

        lax.fori_loop(0, ZROWS // 16, zb, 0)
        pltpu.sync_copy(zbuf, sh_deg.at[pl.ds(sid * ZROWS, ZROWS)])
        wp.wait()
        plsc.subcore_barrier()

        def group(g, _):
            for b in range(NB):
                k = g * NB + b

                @pl.when(g > 0)
                def _():
                    pltpu.make_async_copy(
                        w_v.at[pl.ds(0, CH)], sh_deg.at[colb[b]], ssem[b]
                    ).wait()

                coff = pl.multiple_of((base + k) * CH, CH)
                pltpu.async_copy(
                    col_hbm.at[pl.ds(coff, CH)], colb[b], csem[b])
            for b in range(NB):
                k = g * NB + b
                pltpu.make_async_copy(
                    col_hbm.at[pl.ds(0, CH)], colb[b], csem[b]).wait()
                off = pl.multiple_of(k * CH, CH)
                pltpu.async_copy(
                    w_v.at[pl.ds(off, CH)], sh_deg.at[colb[b]], ssem[b],
                    add=True)
            return 0

        lax.fori_loop(0, cpw // NB, group, 0)
        for b in range(NB):
            pltpu.make_async_copy(
                w_v.at[pl.ds(0, CH)], sh_deg.at[colb[b]], ssem[b]).wait()
        plsc.subcore_barrier()
        pltpu.sync_copy(
            sh_deg.at[pl.ds(sid * ZROWS, ZROWS)],
            out_hbm.at[cid, pl.ds(sid * ZROWS, ZROWS)],
        )

    NA = 2

    @functools.partial(
        pl.kernel,
        out_type=jax.ShapeDtypeStruct((NC, NP, H), jnp.float32),
        mesh=mesh,
        compiler_params=sc_params,
        scratch_types=[
            [pltpu.VMEM((CH,), jnp.int32)] * NA,
            [pltpu.VMEM((CH,), jnp.int32)] * NA,
            [pltpu.VMEM((CH,), jnp.float32)] * NA,
            [pltpu.VMEM((CH, H), jnp.float32)] * NA,
            pltpu.VMEM((16, H), jnp.float32),
            pltpu.VMEM_SHARED((NP, H), jnp.float32),
            [pltpu.SemaphoreType.DMA] * NA,
            [pltpu.SemaphoreType.DMA] * NA,
            [pltpu.SemaphoreType.DMA] * NA,
            [pltpu.SemaphoreType.DMA] * NA,
            [pltpu.SemaphoreType.DMA] * NA,
        ],
    )
    def agg_kernel(x_hbm, rowf_hbm, col_hbm, norm_hbm, out_hbm,
                   rowb, colb, normb, rowsb, zb, sh_agg,
                   rsem, csem, nsem, gsem, ssem):
        cid = lax.axis_index("c")
        sid = lax.axis_index("s")
        base = cid * (nchunk // NC) + sid * cpw

        def zloop(i, _):
            for q in range(H // 16):
                zb[i, pl.ds(q * 16, 16)] = jnp.zeros((16,), jnp.float32)
            return 0

        lax.fori_loop(0, 16, zloop, 0)

        def zc(i, _):
            pltpu.sync_copy(zb, sh_agg.at[pl.ds(sid * ZROWS + i * 16, 16)])
            return 0

        lax.fori_loop(0, ZROWS // 16, zc, 0)
        plsc.subcore_barrier()

        def do_chunks(ks, nb, first):
            for b in range(nb):
                k = ks + b
                if not first:
                    pltpu.make_async_copy(
                        rowsb[b], sh_agg.at[colb[b]], ssem[b]).wait()
                coff = pl.multiple_of((base + k) * CH, CH)
                pltpu.async_copy(
                    rowf_hbm.at[pl.ds(coff, CH)], rowb[b], rsem[b])
                pltpu.async_copy(
                    col_hbm.at[pl.ds(coff, CH)], colb[b], csem[b])
                pltpu.async_copy(
                    norm_hbm.at[pl.ds(coff, CH)], normb[b], nsem[b])
            for b in range(nb):
                pltpu.make_async_copy(
                    rowf_hbm.at[pl.ds(0, CH)], rowb[b], rsem[b]).wait()
                pltpu.async_copy(x_hbm.at[rowb[b]], rowsb[b], gsem[b])
            for b in range(nb):
                pltpu.make_async_copy(
                    x_hbm.at[rowb[b]], rowsb[b], gsem[b]).wait()
                pltpu.make_async_copy(
                    norm_hbm.at[pl.ds(0, CH)], normb[b], nsem[b]).wait()

                def scale(jb, _):
                    nvec = normb[b][pl.ds(jb * 16, 16)]
                    for l in range(16):
                        s_ = nvec[l]
                        r = jb * 16 + l
                        for q in range(H // 16):
                            rowsb[b][r, pl.ds(q * 16, 16)] = (
                                rowsb[b][r, pl.ds(q * 16, 16)] * s_)
                    return 0

                lax.fori_loop(0, CH // 16, scale, 0)
                pltpu.make_async_copy(
                    col_hbm.at[pl.ds(0, CH)], colb[b], csem[b]).wait()
                pltpu.async_copy(
                    rowsb[b], sh_agg.at[colb[b]], ssem[b], add=True)

        def group(g, _):
            do_chunks(g * NA, NA, False)
            return 0

        do_chunks(0, NA, True)
        lax.fori_loop(1, cpw // NA, group, 0)
        if cpw % NA:
            do_chunks((cpw // NA) * NA, cpw % NA, False)

        for b in range(NA):
            pltpu.make_async_copy(rowsb[b], sh_agg.at[colb[b]], ssem[b]).wait()
        plsc.subcore_barrier()
        pltpu.sync_copy(
            sh_agg.at[pl.ds(sid * ZROWS, ZROWS)],
            out_hbm.at[cid, pl.ds(sid * ZROWS, ZROWS)],
        )

    return deg_kernel, agg_kernel


def _prep(degpT, x):
    def body(p_ref, x_ref, dis_ref, xs_ref):
        d = p_ref[...]
        dis = lax.rsqrt(d[:, 0:1] + d[:, 1:2])
        dis_ref[...] = dis
        xs_ref[...] = x_ref[...] * dis[:N]

    return pl.pallas_call(
        body,
        out_shape=(
            jax.ShapeDtypeStruct((NP, 1), jnp.float32),
            jax.ShapeDtypeStruct((N, H), jnp.float32),
        ),
    )(degpT, x)


def _gcn_mm(aggp, dis, x0, w, alpha, scale_out):
    def body(p_ref, d_ref, x0_ref, w_ref, o_ref):
        d = d_ref[...][:N]
        agg = (p_ref[0, :N, :] + p_ref[1, :N, :]) * d
        h = (1.0 - alpha) * agg + alpha * x0_ref[...]
        o = jnp.maximum(
            jnp.dot(h, w_ref[...], preferred_element_type=jnp.float32), 0.0
        )
        if scale_out:
            o = o * d
        o_ref[...] = o

    return pl.pallas_call(
        body,
        out_shape=jax.ShapeDtypeStruct((N, H), jnp.float32),
    )(aggp, dis, x0, w)


def _mm_pool(aggp, dis, x0, w, alpha, batch2d, gate_w, gate_b2, nn_w, nn_b2):
    def body(p_ref, d_ref, x0_ref, w_ref, b_ref, gw_ref, gb_ref, nw_ref,
             nb_ref, o_ref):
        d = d_ref[...][:N]
        agg = (p_ref[0, :N, :] + p_ref[1, :N, :]) * d
        hh = (1.0 - alpha) * agg + alpha * x0_ref[...]
        h_ = jnp.maximum(
            jnp.dot(hh, w_ref[...], preferred_element_type=jnp.float32), 0.0
        )
        gT = lax.dot_general(
            gw_ref[...], h_, (((0,), (1,)), ((), ())),
            preferred_element_type=jnp.float32,
        ) + gb_ref[0, 0]
        v = jnp.dot(h_, nw_ref[...], preferred_element_type=jnp.float32)
        v = v + nb_ref[...]
        bat = b_ref[...]
        seg = lax.broadcasted_iota(jnp.int32, (G, N), 0)
        mask = seg == bat
        gbig = jnp.broadcast_to(gT, (G, N))
        m = jnp.max(jnp.where(mask, gbig, -1e30), axis=1, keepdims=True)
        m_n = jnp.sum(jnp.where(mask, jnp.broadcast_to(m, (G, N)), 0.0),
                      axis=0, keepdims=True)
        g = jnp.exp(gT - m_n)
        s = jnp.sum(jnp.where(mask, jnp.broadcast_to(g, (G, N)), 0.0),
                    axis=1, keepdims=True)
        s_n = jnp.sum(jnp.where(mask, jnp.broadcast_to(s, (G, N)), 0.0),
                      axis=0, keepdims=True)
        wn = g / (s_n + 1e-16)
        wm = jnp.where(mask, jnp.broadcast_to(wn, (G, N)), 0.0)
        o_ref[...] = lax.dot_general(
            wm, v, (((1,), (0,)), ((), ())),
            preferred_element_type=jnp.float32,
        )

    return pl.pallas_call(
        body,
        out_shape=jax.ShapeDtypeStruct((G, H), jnp.float32),
    )(aggp, dis, x0, w, batch2d, gate_w, gate_b2, nn_w, nn_b2)


def kernel(x, edge_index, edge_attr, batch, w1, w2, w3, gate_w, gate_b, nn_w, nn_b):
    n = x.shape[0]
    e = edge_index.shape[1]
    et = e + n
    cpw = -(-et // (NW * CH * NB)) * NB
    nchunk = NW * cpw
    ep = nchunk * CH

    loop = jnp.arange(n, dtype=jnp.int32)
    pad = jnp.arange(ep - et, dtype=jnp.int32) % n
    rowf = jnp.concatenate([edge_index[0], loop, pad])
    colf = jnp.concatenate([edge_index[1], loop, pad])
    wf = jnp.concatenate(
        [edge_attr, jnp.ones((n,), jnp.float32),
         jnp.zeros((ep - et,), jnp.float32)]
    )

    deg_kernel, agg_kernel = _make_sc_kernels(nchunk, cpw)

    degp = deg_kernel(colf, wf)
    dis, xs = _prep(degp.T, x)

    aggp1 = agg_kernel(xs, rowf, colf, wf)
    h1 = _gcn_mm(aggp1, dis, x, w1, 0.2, True)
    aggp2 = agg_kernel(h1, rowf, colf, wf)
    h2 = _gcn_mm(aggp2, dis, x, w2, 0.2, True)
    aggp3 = agg_kernel(h2, rowf, colf, wf)

    batch2d = batch.reshape(1, n).astype(jnp.int32)
    gate_b2 = gate_b.reshape(1, 1)
    nn_b2 = nn_b.reshape(1, H)
    return _mm_pool(aggp3, dis, x, w3, 0.4, batch2d, gate_w, gate_b2,
                    nn_w, nn_b2)
